# baseline (speedup 1.0000x reference)
import functools

import jax
import jax.numpy as jnp
from jax import lax
from jax.experimental import pallas as pl
from jax.experimental.pallas import tpu as pltpu
from jax.experimental.pallas import tpu_sc as plsc

VOCAB = 100000
EMBED = 64
BATCH = 4096
TP = 8

_info = plsc.get_sparse_core_info()
_NC, _NS, _L = _info.num_cores, _info.num_subcores, _info.num_lanes
_NW = _NC * _NS
_CHUNKS = BATCH // TP
_WPT = _NW // TP
_CPW = _CHUNKS // _WPT


def _sc_body(x_hbm, table_hbm, out_hbm, posbuf, idxbuf, rows, sem):
    wid = lax.axis_index("s") * _NC + lax.axis_index("c")
    t = wid // _WPT
    cbase = (wid % _WPT) * _CPW
    lanes = lax.iota(jnp.int32, _L)
    base = cbase * TP + t
    for j in range(_CPW // _L):
        posbuf[pl.ds(j * _L, _L)] = (j * _L + lanes) * TP + base
    pltpu.async_copy(x_hbm.at[posbuf], idxbuf, sem).wait()
    pltpu.async_copy(table_hbm.at[idxbuf], rows, sem).wait()
    pltpu.sync_copy(rows, out_hbm.at[t, pl.ds(cbase, _CPW)])


_gather_embed = functools.partial(
    pl.kernel,
    out_type=jax.ShapeDtypeStruct((TP, _CHUNKS, EMBED), jnp.float32),
    mesh=plsc.VectorSubcoreMesh(core_axis_name="c", subcore_axis_name="s"),
    scratch_types=[
        pltpu.VMEM((_CPW,), jnp.int32),
        pltpu.VMEM((_CPW,), jnp.int32),
        pltpu.VMEM((_CPW, EMBED), jnp.float32),
        pltpu.SemaphoreType.DMA,
    ],
    compiler_params=pltpu.CompilerParams(use_tc_tiling_on_sc=False),
)(_sc_body)


@jax.jit
def kernel(x, table):
    out = _gather_embed(jnp.asarray(x, jnp.int32), table)
    return out.reshape(TP, _CHUNKS * EMBED)
